# baseline (speedup 1.0000x reference)
import functools

import jax
import jax.numpy as jnp
from jax import lax
from jax.experimental import pallas as pl
from jax.experimental.pallas import tpu as pltpu
from jax.experimental.pallas import tpu_sc as plsc

NC = 2
NS = 16
NW = NC * NS
CHUNK = 128

_HIGH = lax.Precision.HIGHEST


def _cdiv(a, b):
    return -(-a // b)


def _make_deg_kernel(npad, ch_per_w):
    def body(dst_hbm, out_hbm, dst_v, degbuf):
        c = lax.axis_index("c")
        s = lax.axis_index("s")
        w = s * NC + c
        start = w * ch_per_w
        z16 = jnp.zeros((16,), jnp.float32)
        o16 = jnp.ones((16,), jnp.float32)

        def fill(i, _):
            degbuf[pl.ds(i * 16, 16)] = z16
            return 0

        lax.fori_loop(0, npad // 16, fill, 0)
        pltpu.sync_copy(dst_hbm.at[pl.ds(start, ch_per_w)], dst_v)

        def edge_chunk(j, _):
            for k in range(CHUNK // 16):
                idx = dst_v[j, pl.ds(k * 16, 16)]
                plsc.addupdate_scatter(degbuf, [idx], o16)
            return 0

        lax.fori_loop(0, ch_per_w, edge_chunk, 0)
        pltpu.sync_copy(degbuf, out_hbm.at[w])

    mesh = plsc.VectorSubcoreMesh(core_axis_name="c", subcore_axis_name="s")
    return pl.kernel(
        body,
        out_type=jax.ShapeDtypeStruct((NW, npad), jnp.float32),
        mesh=mesh,
        scratch_types=[
            pltpu.VMEM((ch_per_w, CHUNK), jnp.int32),
            pltpu.VMEM((npad,), jnp.float32),
        ],
        compiler_params=pltpu.CompilerParams(needs_layout_passes=False),
    )


def _make_agg_kernel(npad, d, ch_light, ch_heavy, light_c):
    rows_per_tile = npad // NS
    copy_sizes = [CHUNK] * (rows_per_tile // CHUNK)
    if rows_per_tile % CHUNK:
        copy_sizes.append(rows_per_tile % CHUNK)

    NBUF = 2

    def body(hp_hbm, src_hbm, dst_hbm, out_hbm, srcb, dstb,
             b0, b1, acc, g0, g1):
        bufs = [b0, b1]
        gs = [g0, g1]
        c = lax.axis_index("c")
        s = lax.axis_index("s")
        w = s * NC + c
        is_light = c == light_c
        start = jnp.where(is_light, s * ch_light,
                          NS * ch_light + s * ch_heavy)
        cnt = jnp.where(is_light, ch_light, ch_heavy)
        rounds = cnt // NBUF
        z16 = jnp.zeros((16,), jnp.float32)

        def fill(i, _):
            for k in range(d // 16):
                b0[i, pl.ds(k * 16, 16)] = z16
            return 0

        lax.fori_loop(0, CHUNK, fill, 0)
        base = s * rows_per_tile
        off = 0
        for sz in copy_sizes:
            pltpu.sync_copy(b0.at[pl.ds(0, sz)],
                            acc.at[pl.ds(base + off, sz)])
            off += sz
        plsc.subcore_barrier()

        for b in range(NBUF):
            pltpu.sync_copy(src_hbm.at[start + b], srcb.at[b])
            pltpu.sync_copy(dst_hbm.at[start + b], dstb.at[b])
            pltpu.async_copy(hp_hbm.at[srcb.at[b]], bufs[b], gs[b])

        def round_body(i, _):
            for b in range(NBUF):
                j = NBUF * i + b
                pltpu.make_async_copy(
                    hp_hbm.at[srcb.at[b]], bufs[b], gs[b]).wait()
                pltpu.sync_copy(bufs[b], acc.at[dstb.at[b]], add=True)

                @pl.when(i < rounds - 1)
                def _():
                    pltpu.sync_copy(src_hbm.at[start + j + NBUF], srcb.at[b])
                    pltpu.sync_copy(dst_hbm.at[start + j + NBUF], dstb.at[b])
                    pltpu.async_copy(hp_hbm.at[srcb.at[b]], bufs[b], gs[b])
            return 0

        lax.fori_loop(0, rounds, round_body, 0)
        plsc.subcore_barrier()
        off = 0
        for sz in copy_sizes:
            pltpu.sync_copy(acc.at[pl.ds(base + off, sz)],
                            b0.at[pl.ds(0, sz)])
            pltpu.sync_copy(b0.at[pl.ds(0, sz)],
                            out_hbm.at[w, pl.ds(off, sz)])
            off += sz

    mesh = plsc.VectorSubcoreMesh(core_axis_name="c", subcore_axis_name="s")
    return pl.kernel(
        body,
        out_type=jax.ShapeDtypeStruct((NW, rows_per_tile, d), jnp.float32),
        mesh=mesh,
        scratch_types=[
            pltpu.VMEM((NBUF, CHUNK), jnp.int32),
            pltpu.VMEM((NBUF, CHUNK), jnp.int32),
            pltpu.VMEM((CHUNK, d), jnp.float32),
            pltpu.VMEM((CHUNK, d), jnp.float32),
            pltpu.VMEM_SHARED((npad, d), jnp.float32),
            pltpu.SemaphoreType.DMA,
            pltpu.SemaphoreType.DMA,
        ],
    )


def _prep_call(dacc, xpad, w0, bn):
    npad, d = xpad.shape
    h = w0.shape[1]
    nb = npad // bn

    def body(dacc_ref, x_ref, w_ref, hp_ref, dis_ref):
        deg = jnp.sum(dacc_ref[...], axis=1, keepdims=True) + 1.0
        dis = lax.rsqrt(deg)
        hm = jnp.dot(x_ref[...], w_ref[...],
                     preferred_element_type=jnp.float32, precision=_HIGH)
        hp_ref[...] = dis * hm
        dis_ref[...] = dis

    return pl.pallas_call(
        body,
        grid=(nb,),
        in_specs=[
            pl.BlockSpec((bn, NW), lambda j: (j, 0)),
            pl.BlockSpec((bn, d), lambda j: (j, 0)),
            pl.BlockSpec((d, h), lambda j: (0, 0)),
        ],
        out_specs=[
            pl.BlockSpec((bn, h), lambda j: (j, 0)),
            pl.BlockSpec((bn, 1), lambda j: (j, 0)),
        ],
        out_shape=[
            jax.ShapeDtypeStruct((npad, h), jnp.float32),
            jax.ShapeDtypeStruct((npad, 1), jnp.float32),
        ],
    )(dacc, xpad, w0)


def _post_call(sacc4, hp, dis, b, wn, xres, bn):
    npad, h = hp.shape
    rpt = sacc4.shape[2]
    hn = wn.shape[1]
    nb = npad // bn
    has_res = xres is not None

    def body(sacc_ref, hp_ref, dis_ref, b_ref, w_ref, *rest):
        if has_res:
            xres_ref, xout_ref, hn_ref = rest
        else:
            xout_ref, hn_ref = rest
        t = sacc_ref[0, 0] + sacc_ref[0, 1] + hp_ref[...]
        y = jnp.maximum(dis_ref[...] * t + b_ref[...], 0.0)
        xout_ref[...] = y
        xin = y + xres_ref[...] if has_res else y
        hm = jnp.dot(xin, w_ref[...],
                     preferred_element_type=jnp.float32, precision=_HIGH)
        hn_ref[...] = dis_ref[...] * hm

    in_specs = [
        pl.BlockSpec((1, NC, rpt, h), lambda j: (j, 0, 0, 0)),
        pl.BlockSpec((bn, h), lambda j: (j, 0)),
        pl.BlockSpec((bn, 1), lambda j: (j, 0)),
        pl.BlockSpec((1, h), lambda j: (0, 0)),
        pl.BlockSpec((h, hn), lambda j: (0, 0)),
    ]
    args = [sacc4, hp, dis, b, wn]
    if has_res:
        in_specs.append(pl.BlockSpec((bn, h), lambda j: (j, 0)))
        args.append(xres)
    return pl.pallas_call(
        body,
        grid=(nb,),
        in_specs=in_specs,
        out_specs=[
            pl.BlockSpec((bn, h), lambda j: (j, 0)),
            pl.BlockSpec((bn, hn), lambda j: (j, 0)),
        ],
        out_shape=[
            jax.ShapeDtypeStruct((npad, h), jnp.float32),
            jax.ShapeDtypeStruct((npad, hn), jnp.float32),
        ],
    )(*args)


def _final_call(sacc4, hp, dis, b, batch2d, wl, bl, g, bn):
    npad, h = hp.shape
    rpt = sacc4.shape[2]
    c = wl.shape[1]
    nb = npad // bn

    def body(sacc_ref, hp_ref, dis_ref, b_ref, batch_ref, wl_ref, bl_ref,
             y_ref, gm_ref, gsum, gcnt):
        j = pl.program_id(0)
        t = sacc_ref[0, 0] + sacc_ref[0, 1] + hp_ref[...]
        x3 = jnp.maximum(dis_ref[...] * t + b_ref[...], 0.0)
        gids = lax.broadcasted_iota(jnp.int32, (1, g), 1)
        onehot = (batch_ref[...] == gids).astype(jnp.float32)
        part = lax.dot_general(onehot, x3, (((0,), (0,)), ((), ())),
                               preferred_element_type=jnp.float32,
                               precision=_HIGH)
        ones_col = jnp.ones((bn, 1), jnp.float32)
        cpart = lax.dot_general(onehot, ones_col, (((0,), (0,)), ((), ())),
                                preferred_element_type=jnp.float32,
                                precision=_HIGH)

        @pl.when(j == 0)
        def _():
            gsum[...] = part
            gcnt[...] = cpart

        @pl.when(j > 0)
        def _():
            gsum[...] += part
            gcnt[...] += cpart

        @pl.when(j == nb - 1)
        def _():
            gm = gsum[...] / jnp.maximum(gcnt[...], 1.0)
            gm_ref[...] = gm
            y_ref[...] = jnp.dot(gm, wl_ref[...],
                                 preferred_element_type=jnp.float32,
                                 precision=_HIGH) + bl_ref[...]

    return pl.pallas_call(
        body,
        grid=(nb,),
        in_specs=[
            pl.BlockSpec((1, NC, rpt, h), lambda j: (j, 0, 0, 0)),
            pl.BlockSpec((bn, h), lambda j: (j, 0)),
            pl.BlockSpec((bn, 1), lambda j: (j, 0)),
            pl.BlockSpec((1, h), lambda j: (0, 0)),
            pl.BlockSpec((bn, 1), lambda j: (j, 0)),
            pl.BlockSpec((h, c), lambda j: (0, 0)),
            pl.BlockSpec((1, c), lambda j: (0, 0)),
        ],
        out_specs=[
            pl.BlockSpec((g, c), lambda j: (0, 0)),
            pl.BlockSpec((g, h), lambda j: (0, 0)),
        ],
        out_shape=[
            jax.ShapeDtypeStruct((g, c), jnp.float32),
            jax.ShapeDtypeStruct((g, h), jnp.float32),
        ],
        scratch_shapes=[
            pltpu.VMEM((g, h), jnp.float32),
            pltpu.VMEM((g, 1), jnp.float32),
        ],
    )(sacc4, hp, dis, b, batch2d, wl, bl)


def kernel(x, edge_index, batch, W0, b0, W1, b1, W2, b2, Wl, bl):
    n, d = x.shape
    h = W0.shape[1]
    c_out = Wl.shape[1]
    g = 64
    e = edge_index.shape[1]

    ct = _cdiv(e, CHUNK)
    ch_light = 16 * max(1, round(ct * 0.3 / NS / 16))
    ch_heavy = 16 * _cdiv(max(ct - NS * ch_light, NS * 16), NS * 16)
    ct_pad = NS * (ch_light + ch_heavy)
    e_pad = ct_pad * CHUNK
    light_c = 1
    npad = _cdiv(n + 1, NS * 8) * NS * 8
    rpt = npad // NS
    bn = rpt

    src = edge_index[0].astype(jnp.int32)
    dst = edge_index[1].astype(jnp.int32)
    pad = e_pad - e
    src3 = jnp.concatenate([src, jnp.zeros((pad,), jnp.int32)]
                           ).reshape(ct_pad, CHUNK)
    dst3 = jnp.concatenate([dst, jnp.full((pad,), n, jnp.int32)]
                           ).reshape(ct_pad, CHUNK)
    xpad = jnp.pad(x, ((0, npad - n), (0, 0)))
    batch2d = jnp.pad(batch.astype(jnp.int32), (0, npad - n),
                      constant_values=-1).reshape(npad, 1)
    b0r = b0.reshape(1, h)
    b1r = b1.reshape(1, h)
    b2r = b2.reshape(1, h)
    blr = bl.reshape(1, c_out)

    deg_k = _make_deg_kernel(npad, ct_pad // NW)
    agg_k = _make_agg_kernel(npad, h, ch_light, ch_heavy, light_c)

    def agg4(hp):
        return agg_k(hp, src3, dst3).reshape(NS, NC, rpt, h)

    dacc = jnp.transpose(deg_k(dst3))
    h0p, dis = _prep_call(dacc, xpad, W0, bn)
    s1 = agg4(h0p)
    x1, h1p = _post_call(s1, h0p, dis, b0r, W1, None, bn)
    s2 = agg4(h1p)
    _, h2p = _post_call(s2, h1p, dis, b1r, W2, x1, bn)
    s3 = agg4(h2p)
    y, gm = _final_call(s3, h2p, dis, b2r, batch2d, Wl, blr, g, bn)
    return (y, gm)


# speedup vs baseline: 1.0403x; 1.0403x over previous
import functools

import jax
import jax.numpy as jnp
from jax import lax
from jax.experimental import pallas as pl
from jax.experimental.pallas import tpu as pltpu
from jax.experimental.pallas import tpu_sc as plsc

NC = 2
NS = 16
NW = NC * NS
CHUNK = 128

_HIGH = lax.Precision.HIGHEST


def _cdiv(a, b):
    return -(-a // b)


def _make_deg_kernel(npad, ch_per_w):
    def body(dst_hbm, out_hbm, dst_v, degbuf):
        c = lax.axis_index("c")
        s = lax.axis_index("s")
        w = s * NC + c
        start = w * ch_per_w
        z16 = jnp.zeros((16,), jnp.float32)
        o16 = jnp.ones((16,), jnp.float32)

        def fill(i, _):
            degbuf[pl.ds(i * 16, 16)] = z16
            return 0

        lax.fori_loop(0, npad // 16, fill, 0)
        pltpu.sync_copy(dst_hbm.at[pl.ds(start, ch_per_w)], dst_v)

        def edge_chunk(j, _):
            for k in range(CHUNK // 16):
                idx = dst_v[j, pl.ds(k * 16, 16)]
                plsc.addupdate_scatter(degbuf, [idx], o16)
            return 0

        lax.fori_loop(0, ch_per_w, edge_chunk, 0)
        pltpu.sync_copy(degbuf, out_hbm.at[w])

    mesh = plsc.VectorSubcoreMesh(core_axis_name="c", subcore_axis_name="s")
    return pl.kernel(
        body,
        out_type=jax.ShapeDtypeStruct((NW, npad), jnp.float32),
        mesh=mesh,
        scratch_types=[
            pltpu.VMEM((ch_per_w, CHUNK), jnp.int32),
            pltpu.VMEM((npad,), jnp.float32),
        ],
        compiler_params=pltpu.CompilerParams(needs_layout_passes=False),
    )


def _make_agg_kernel(npad, d, ch_light, ch_heavy, light_c):
    rows_per_tile = npad // NS
    copy_sizes = [CHUNK] * (rows_per_tile // CHUNK)
    if rows_per_tile % CHUNK:
        copy_sizes.append(rows_per_tile % CHUNK)

    NBUF = 2

    def body(hp_hbm, src_hbm, dst_hbm, out_hbm, srcb, dstb,
             b0, b1, acc, g0, g1):
        bufs = [b0, b1]
        gs = [g0, g1]
        c = lax.axis_index("c")
        s = lax.axis_index("s")
        w = s * NC + c
        is_light = c == light_c
        start = jnp.where(is_light, s * ch_light,
                          NS * ch_light + s * ch_heavy)
        cnt = jnp.where(is_light, ch_light, ch_heavy)
        rounds = cnt // NBUF
        z16 = jnp.zeros((16,), jnp.float32)

        def fill(i, _):
            for k in range(d // 16):
                b0[i, pl.ds(k * 16, 16)] = z16
            return 0

        lax.fori_loop(0, CHUNK, fill, 0)
        base = s * rows_per_tile
        off = 0
        for sz in copy_sizes:
            pltpu.sync_copy(b0.at[pl.ds(0, sz)],
                            acc.at[pl.ds(base + off, sz)])
            off += sz
        plsc.subcore_barrier()

        for b in range(NBUF):
            pltpu.sync_copy(src_hbm.at[start + b], srcb.at[b])
            pltpu.sync_copy(dst_hbm.at[start + b], dstb.at[b])
            pltpu.async_copy(hp_hbm.at[srcb.at[b]], bufs[b], gs[b])

        def round_body(i, _):
            for b in range(NBUF):
                j = NBUF * i + b
                pltpu.make_async_copy(
                    hp_hbm.at[srcb.at[b]], bufs[b], gs[b]).wait()
                pltpu.sync_copy(bufs[b], acc.at[dstb.at[b]], add=True)

                @pl.when(i < rounds - 1)
                def _():
                    pltpu.sync_copy(src_hbm.at[start + j + NBUF], srcb.at[b])
                    pltpu.sync_copy(dst_hbm.at[start + j + NBUF], dstb.at[b])
                    pltpu.async_copy(hp_hbm.at[srcb.at[b]], bufs[b], gs[b])
            return 0

        lax.fori_loop(0, rounds, round_body, 0)
        plsc.subcore_barrier()
        off = 0
        for sz in copy_sizes:
            pltpu.sync_copy(acc.at[pl.ds(base + off, sz)],
                            b0.at[pl.ds(0, sz)])
            pltpu.sync_copy(b0.at[pl.ds(0, sz)],
                            out_hbm.at[w, pl.ds(off, sz)])
            off += sz

    mesh = plsc.VectorSubcoreMesh(core_axis_name="c", subcore_axis_name="s")
    return pl.kernel(
        body,
        out_type=jax.ShapeDtypeStruct((NW, rows_per_tile, d), jnp.float32),
        mesh=mesh,
        scratch_types=[
            pltpu.VMEM((NBUF, CHUNK), jnp.int32),
            pltpu.VMEM((NBUF, CHUNK), jnp.int32),
            pltpu.VMEM((CHUNK, d), jnp.float32),
            pltpu.VMEM((CHUNK, d), jnp.float32),
            pltpu.VMEM_SHARED((npad, d), jnp.float32),
            pltpu.SemaphoreType.DMA,
            pltpu.SemaphoreType.DMA,
        ],
    )


def _prep_call(dacc, xpad, w0, bn):
    npad, d = xpad.shape
    h = w0.shape[1]
    nb = npad // bn

    def body(dacc_ref, x_ref, w_ref, hp_ref, dis_ref):
        deg = jnp.sum(dacc_ref[...], axis=1, keepdims=True) + 1.0
        dis = lax.rsqrt(deg)
        hm = jnp.dot(x_ref[...], w_ref[...],
                     preferred_element_type=jnp.float32, precision=_HIGH)
        hp_ref[...] = dis * hm
        dis_ref[...] = dis

    return pl.pallas_call(
        body,
        grid=(nb,),
        in_specs=[
            pl.BlockSpec((bn, NW), lambda j: (j, 0)),
            pl.BlockSpec((bn, d), lambda j: (j, 0)),
            pl.BlockSpec((d, h), lambda j: (0, 0)),
        ],
        out_specs=[
            pl.BlockSpec((bn, h), lambda j: (j, 0)),
            pl.BlockSpec((bn, 1), lambda j: (j, 0)),
        ],
        out_shape=[
            jax.ShapeDtypeStruct((npad, h), jnp.float32),
            jax.ShapeDtypeStruct((npad, 1), jnp.float32),
        ],
    )(dacc, xpad, w0)


def _post_call(sacc4, hp, dis, b, wn, xres, bn):
    npad, h = hp.shape
    rpt = sacc4.shape[2]
    hn = wn.shape[1]
    nb = npad // bn
    has_res = xres is not None

    def body(sacc_ref, hp_ref, dis_ref, b_ref, w_ref, *rest):
        if has_res:
            xres_ref, xout_ref, hn_ref = rest
        else:
            xout_ref, hn_ref = rest
        t = sacc_ref[0, 0] + sacc_ref[0, 1] + hp_ref[...]
        y = jnp.maximum(dis_ref[...] * t + b_ref[...], 0.0)
        xout_ref[...] = y
        xin = y + xres_ref[...] if has_res else y
        hm = jnp.dot(xin, w_ref[...],
                     preferred_element_type=jnp.float32, precision=_HIGH)
        hn_ref[...] = dis_ref[...] * hm

    in_specs = [
        pl.BlockSpec((1, NC, rpt, h), lambda j: (j, 0, 0, 0)),
        pl.BlockSpec((bn, h), lambda j: (j, 0)),
        pl.BlockSpec((bn, 1), lambda j: (j, 0)),
        pl.BlockSpec((1, h), lambda j: (0, 0)),
        pl.BlockSpec((h, hn), lambda j: (0, 0)),
    ]
    args = [sacc4, hp, dis, b, wn]
    if has_res:
        in_specs.append(pl.BlockSpec((bn, h), lambda j: (j, 0)))
        args.append(xres)
    return pl.pallas_call(
        body,
        grid=(nb,),
        in_specs=in_specs,
        out_specs=[
            pl.BlockSpec((bn, h), lambda j: (j, 0)),
            pl.BlockSpec((bn, hn), lambda j: (j, 0)),
        ],
        out_shape=[
            jax.ShapeDtypeStruct((npad, h), jnp.float32),
            jax.ShapeDtypeStruct((npad, hn), jnp.float32),
        ],
    )(*args)


def _final_call(sacc4, hp, dis, b, batch2d, wl, bl, g, bn):
    npad, h = hp.shape
    rpt = sacc4.shape[2]
    c = wl.shape[1]
    nb = npad // bn

    def body(sacc_ref, hp_ref, dis_ref, b_ref, batch_ref, wl_ref, bl_ref,
             y_ref, gm_ref, gsum, gcnt):
        j = pl.program_id(0)
        t = sacc_ref[0, 0] + sacc_ref[0, 1] + hp_ref[...]
        x3 = jnp.maximum(dis_ref[...] * t + b_ref[...], 0.0)
        gids = lax.broadcasted_iota(jnp.int32, (1, g), 1)
        onehot = (batch_ref[...] == gids).astype(jnp.float32)
        part = lax.dot_general(onehot, x3, (((0,), (0,)), ((), ())),
                               preferred_element_type=jnp.float32,
                               precision=_HIGH)
        ones_col = jnp.ones((bn, 1), jnp.float32)
        cpart = lax.dot_general(onehot, ones_col, (((0,), (0,)), ((), ())),
                                preferred_element_type=jnp.float32,
                                precision=_HIGH)

        @pl.when(j == 0)
        def _():
            gsum[...] = part
            gcnt[...] = cpart

        @pl.when(j > 0)
        def _():
            gsum[...] += part
            gcnt[...] += cpart

        @pl.when(j == nb - 1)
        def _():
            gm = gsum[...] / jnp.maximum(gcnt[...], 1.0)
            gm_ref[...] = gm
            y_ref[...] = jnp.dot(gm, wl_ref[...],
                                 preferred_element_type=jnp.float32,
                                 precision=_HIGH) + bl_ref[...]

    return pl.pallas_call(
        body,
        grid=(nb,),
        in_specs=[
            pl.BlockSpec((1, NC, rpt, h), lambda j: (j, 0, 0, 0)),
            pl.BlockSpec((bn, h), lambda j: (j, 0)),
            pl.BlockSpec((bn, 1), lambda j: (j, 0)),
            pl.BlockSpec((1, h), lambda j: (0, 0)),
            pl.BlockSpec((bn, 1), lambda j: (j, 0)),
            pl.BlockSpec((h, c), lambda j: (0, 0)),
            pl.BlockSpec((1, c), lambda j: (0, 0)),
        ],
        out_specs=[
            pl.BlockSpec((g, c), lambda j: (0, 0)),
            pl.BlockSpec((g, h), lambda j: (0, 0)),
        ],
        out_shape=[
            jax.ShapeDtypeStruct((g, c), jnp.float32),
            jax.ShapeDtypeStruct((g, h), jnp.float32),
        ],
        scratch_shapes=[
            pltpu.VMEM((g, h), jnp.float32),
            pltpu.VMEM((g, 1), jnp.float32),
        ],
    )(sacc4, hp, dis, b, batch2d, wl, bl)


def kernel(x, edge_index, batch, W0, b0, W1, b1, W2, b2, Wl, bl):
    n, d = x.shape
    h = W0.shape[1]
    c_out = Wl.shape[1]
    g = 64
    e = edge_index.shape[1]

    ct = _cdiv(e, CHUNK)
    ch_light = 16 * max(1, round(ct * 0.5 / NS / 16))
    ch_heavy = 16 * _cdiv(max(ct - NS * ch_light, NS * 16), NS * 16)
    ct_pad = NS * (ch_light + ch_heavy)
    e_pad = ct_pad * CHUNK
    light_c = 0
    npad = _cdiv(n + 1, NS * 8) * NS * 8
    rpt = npad // NS
    bn = rpt

    src = edge_index[0].astype(jnp.int32)
    dst = edge_index[1].astype(jnp.int32)
    pad = e_pad - e
    src3 = jnp.concatenate([src, jnp.zeros((pad,), jnp.int32)]
                           ).reshape(ct_pad, CHUNK)
    dst3 = jnp.concatenate([dst, jnp.full((pad,), n, jnp.int32)]
                           ).reshape(ct_pad, CHUNK)
    xpad = jnp.pad(x, ((0, npad - n), (0, 0)))
    batch2d = jnp.pad(batch.astype(jnp.int32), (0, npad - n),
                      constant_values=-1).reshape(npad, 1)
    b0r = b0.reshape(1, h)
    b1r = b1.reshape(1, h)
    b2r = b2.reshape(1, h)
    blr = bl.reshape(1, c_out)

    deg_k = _make_deg_kernel(npad, ct_pad // NW)
    agg_k = _make_agg_kernel(npad, h, ch_light, ch_heavy, light_c)

    def agg4(hp):
        return agg_k(hp, src3, dst3).reshape(NS, NC, rpt, h)

    dacc = jnp.transpose(deg_k(dst3))
    h0p, dis = _prep_call(dacc, xpad, W0, bn)
    s1 = agg4(h0p)
    x1, h1p = _post_call(s1, h0p, dis, b0r, W1, None, bn)
    s2 = agg4(h1p)
    _, h2p = _post_call(s2, h1p, dis, b1r, W2, x1, bn)
    s3 = agg4(h2p)
    y, gm = _final_call(s3, h2p, dis, b2r, batch2d, Wl, blr, g, bn)
    return (y, gm)
